# baseline (speedup 1.0000x reference)
import functools

import jax
import jax.numpy as jnp
from jax import lax
from jax.experimental import pallas as pl
from jax.experimental.pallas import tpu as pltpu
from jax.experimental.pallas import tpu_sc as plsc

NUM_CORES = 2
NUM_SUBCORES = 16
NUM_WORKERS = NUM_CORES * NUM_SUBCORES

CHUNK = 128
FEATURES = 32


def _build_sc_gather(total_rows: int, features: int, table_rows: int):
    assert total_rows % (NUM_WORKERS * CHUNK) == 0
    rows_per_w = total_rows // NUM_WORKERS
    chunks_per_w = rows_per_w // CHUNK

    mesh = plsc.VectorSubcoreMesh(
        core_axis_name="c", subcore_axis_name="s",
        num_cores=NUM_CORES, num_subcores=NUM_SUBCORES)

    SLAB = 1280
    nbuf = 2
    assert rows_per_w % SLAB == 0
    n_slabs = rows_per_w // SLAB

    @functools.partial(
        pl.kernel,
        out_type=jax.ShapeDtypeStruct(
            (total_rows // SLAB, SLAB, features), jnp.float32),
        mesh=mesh,
        scratch_types=[
            pltpu.VMEM((n_slabs, SLAB), jnp.int32),
            pltpu.VMEM((nbuf, SLAB, features), jnp.float32),
            [pltpu.SemaphoreType.DMA] * nbuf,
            [pltpu.SemaphoreType.DMA] * nbuf,
        ],
        compiler_params=pltpu.CompilerParams(use_tc_tiling_on_sc=False),
    )
    def sc_gather(idx_hbm, tab_hbm, out_hbm, idx_v, buf, gsems, wsems):
        wid = lax.axis_index("s") * NUM_CORES + lax.axis_index("c")
        slab0 = wid * n_slabs
        pltpu.sync_copy(idx_hbm.at[wid], idx_v)

        for b in range(min(nbuf, n_slabs)):
            pltpu.async_copy(tab_hbm.at[idx_v.at[b]], buf.at[b], gsems[b])

        @pl.loop(0, n_slabs)
        def _(j):
            for b in range(nbuf):

                @pl.when(lax.rem(j, nbuf) == b)
                def _():
                    pltpu.make_async_copy(
                        tab_hbm.at[idx_v.at[0]], buf.at[b], gsems[b]).wait()
                    pltpu.async_copy(buf.at[b], out_hbm.at[slab0 + j],
                                     wsems[b])
                    nj = j + nbuf

                    @pl.when(nj < n_slabs)
                    def _():
                        pltpu.make_async_copy(
                            buf.at[b], out_hbm.at[slab0], wsems[b]).wait()
                        pltpu.async_copy(tab_hbm.at[idx_v.at[nj]],
                                         buf.at[b], gsems[b])

        for b in range(min(nbuf, n_slabs)):
            pltpu.make_async_copy(buf.at[b], out_hbm.at[slab0], wsems[b]).wait()

    return sc_gather


def kernel(inputs, embedding):
    b, s = inputs.shape
    total = b * s
    idx3d = inputs.reshape(NUM_WORKERS, -1, 1280).astype(jnp.int32)
    gather = _build_sc_gather(total, embedding.shape[1], embedding.shape[0])
    out = gather(idx3d, embedding)
    return out.reshape(b, s, embedding.shape[1])
